# baseline (speedup 1.0000x reference)
import functools

import jax
import jax.numpy as jnp
from jax import lax
from jax.experimental import pallas as pl
import jax.experimental.pallas.tpu as pltpu
from jax.experimental.pallas import tpu_sc as plsc

SEQ_LEN = 2048
D_MODEL = 1024
N_REP = 4
NUM_CORES = 2
NUM_SUBCORES = 16
NUM_WORKERS = NUM_CORES * NUM_SUBCORES
CHUNK = SEQ_LEN // NUM_WORKERS

_mesh = plsc.VectorSubcoreMesh(core_axis_name="c", subcore_axis_name="s")


@functools.partial(
    pl.kernel,
    mesh=_mesh,
    out_type=jax.ShapeDtypeStruct((SEQ_LEN, N_REP, D_MODEL), jnp.float32),
    scratch_types=[
        pltpu.VMEM((CHUNK, D_MODEL), jnp.float32),
        pltpu.SemaphoreType.DMA,
    ],
)
def _sc_body(pe_hbm, out_hbm, rows_v, sem):
    wid = lax.axis_index("s") * NUM_CORES + lax.axis_index("c")
    base = wid * CHUNK
    pltpu.sync_copy(pe_hbm.at[pl.ds(base, CHUNK)], rows_v)
    copies = [
        pltpu.async_copy(rows_v, out_hbm.at[pl.ds(base, CHUNK), n], sem)
        for n in range(N_REP)
    ]
    for c in copies:
        c.wait()


def kernel(z, pos_embed):
    del z
    return _sc_body(pos_embed)
